# Initial kernel scaffold; baseline (speedup 1.0000x reference)
import jax
import jax.numpy as jnp
from jax.experimental import pallas as pl


def kernel(E1, edge_index, u, v, n):
    raise NotImplementedError("write your pallas kernel here")


# speedup vs baseline: 17.4402x; 17.4402x over previous
import jax
import jax.numpy as jnp
from jax import lax
from jax.experimental import pallas as pl
from jax.experimental.pallas import tpu as pltpu
from jax.experimental.pallas import tpu_sc as plsc

M = 30000
N = 20000
MN = M + N
DIM = 64
NE = 400000
B = 4096
NNEG = 10
REG = 1e-4

NC = 2
NS = 16
CHUNK = 64
G = 8
NG = 49
CPT = G * NG
PER_CORE = NS * CPT * CHUNK
ACC_ROWS = 30016
DUMMY = ACC_ROWS - 1
ZCH = 32
NZ = ACC_ROWS // ZCH
ZSTEPS = (NZ + NS - 1) // NS
CCH = 40
COPY_STEPS = (M // CCH + NS - 1) // NS

_MESH = plsc.VectorSubcoreMesh(core_axis_name="c", subcore_axis_name="s")
_SC_PARAMS = pltpu.CompilerParams(use_tc_tiling_on_sc=False)


def _fill(ref, val):
    r_, c_ = ref.shape
    vec = jnp.full((16,), val, jnp.float32)

    def row(r, carry):
        for kk in range(c_ // 16):
            ref[r, pl.ds(kk * 16, 16)] = vec
        return carry

    lax.fori_loop(0, r_, row, 0)


def _copy_out(c, s, acc, bounce, out, width):
    limit = jnp.where(c == 0, N // CCH, M // CCH)
    obase = jnp.where(c == 0, M, 0)

    def cpy(jj, carry):
        g = jj * NS + s

        @pl.when(g < limit)
        def _():
            pltpu.sync_copy(acc.at[pl.ds(g * CCH, CCH)], bounce)
            pltpu.sync_copy(bounce, out.at[pl.ds(obase + g * CCH, CCH)])
        return carry

    lax.fori_loop(0, COPY_STEPS, cpy, 0)


def _zero_acc(s, acc, zsrc):
    def z(jj, carry):
        g = jj * NS + s

        @pl.when(g < NZ)
        def _():
            pltpu.sync_copy(zsrc, acc.at[pl.ds(g * ZCH, ZCH)])
        return carry

    lax.fori_loop(0, ZSTEPS, z, 0)


def _make_deg():
    def body(sidx, out, sidx_v, obuf, zbuf, accd):
        c = lax.axis_index("c")
        s = lax.axis_index("s")
        w = c * NS + s
        _fill(obuf, 1.0)
        _fill(zbuf, 0.0)
        _zero_acc(s, accd, zbuf.at[pl.ds(0, ZCH)])
        pltpu.sync_copy(sidx.at[w], sidx_v)
        plsc.subcore_barrier()

        def chunk(j, carry):
            pltpu.sync_copy(obuf, accd.at[sidx_v.at[j]], add=True)
            return carry

        lax.fori_loop(0, CPT, chunk, 0)
        plsc.subcore_barrier()
        _copy_out(c, s, accd, zbuf.at[pl.ds(0, CCH)], out, 16)

    return pl.kernel(
        body,
        out_type=jax.ShapeDtypeStruct((MN, 16), jnp.float32),
        mesh=_MESH,
        compiler_params=_SC_PARAMS,
        scratch_types=[
            pltpu.VMEM((CPT, CHUNK), jnp.int32),
            pltpu.VMEM((CHUNK, 16), jnp.float32),
            pltpu.VMEM((CHUNK, 16), jnp.float32),
            pltpu.VMEM_SHARED((ACC_ROWS, 16), jnp.float32),
        ],
    )


_deg_kernel = _make_deg()


def _make_layer():
    def body(gidx, sidx, table, out,
             gbuf, sbuf, rows_v, acc, sem0, sem1, semi):
        c = lax.axis_index("c")
        s = lax.axis_index("s")
        w = c * NS + s
        _fill(rows_v.at[0], 0.0)
        _zero_acc(s, acc, rows_v.at[0].at[pl.ds(0, ZCH)])
        pltpu.sync_copy(gidx.at[w].at[pl.ds(0, G)], gbuf.at[0])
        pltpu.sync_copy(sidx.at[w].at[pl.ds(0, G)], sbuf.at[0])
        pltpu.async_copy(gidx.at[w].at[pl.ds(G, G)], gbuf.at[1], semi)
        pltpu.async_copy(sidx.at[w].at[pl.ds(G, G)], sbuf.at[1], semi)
        plsc.subcore_barrier()
        pltpu.async_copy(table.at[gbuf.at[0].at[0]], rows_v.at[0], sem0)

        def group(g, carry):
            cur = g % 2
            nxt = 1 - cur
            gb = gbuf.at[cur]
            sb = sbuf.at[cur]

            @pl.when(g + 1 < NG)
            def _():
                pltpu.make_async_copy(gidx.at[w].at[pl.ds(0, G)],
                                      gbuf.at[0], semi).wait()
                pltpu.make_async_copy(gidx.at[w].at[pl.ds(0, G)],
                                      sbuf.at[0], semi).wait()

            for k in range(G):
                j = g * G + k
                b = k % 2
                semb = sem0 if b == 0 else sem1
                semn = sem1 if b == 0 else sem0
                pltpu.make_async_copy(table.at[pl.ds(0, CHUNK)],
                                      rows_v.at[b], semb).wait()
                gnext = gb.at[k + 1] if k + 1 < G else gbuf.at[nxt].at[0]

                @pl.when(j + 1 < CPT)
                def _():
                    pltpu.async_copy(table.at[gnext], rows_v.at[1 - b], semn)
                pltpu.sync_copy(rows_v.at[b], acc.at[sb.at[k]], add=True)

            @pl.when(g + 2 < NG)
            def _():
                pltpu.async_copy(gidx.at[w].at[pl.ds((g + 2) * G, G)],
                                 gbuf.at[cur], semi)
                pltpu.async_copy(sidx.at[w].at[pl.ds((g + 2) * G, G)],
                                 sbuf.at[cur], semi)
            return carry

        lax.fori_loop(0, NG, group, 0)
        plsc.subcore_barrier()
        _copy_out(c, s, acc, rows_v.at[0].at[pl.ds(0, CCH)], out, DIM)

    return pl.kernel(
        body,
        out_type=jax.ShapeDtypeStruct((MN, DIM), jnp.float32),
        mesh=_MESH,
        compiler_params=_SC_PARAMS,
        scratch_types=[
            pltpu.VMEM((2, G, CHUNK), jnp.int32),
            pltpu.VMEM((2, G, CHUNK), jnp.int32),
            pltpu.VMEM((2, CHUNK, DIM), jnp.float32),
            pltpu.VMEM_SHARED((ACC_ROWS, DIM), jnp.float32),
            pltpu.SemaphoreType.DMA,
            pltpu.SemaphoreType.DMA,
            pltpu.SemaphoreType.DMA,
        ],
    )


_layer_kernel = _make_layer()


GB = B * (NNEG + 2)
GPW = GB // (NC * NS)
GCH = 128
GSTEPS = GPW // GCH


def _make_gather():
    def body(idx, emb, out, idx_v, rbuf, sem):
        c = lax.axis_index("c")
        s = lax.axis_index("s")
        w = c * NS + s
        pltpu.sync_copy(idx.at[w], idx_v)

        def chunk(j, carry):
            pltpu.async_copy(emb.at[idx_v.at[j]], rbuf, sem).wait()
            pltpu.sync_copy(rbuf, out.at[pl.ds(w * GPW + j * GCH, GCH)])
            return carry

        lax.fori_loop(0, GSTEPS, chunk, 0)

    return pl.kernel(
        body,
        out_type=jax.ShapeDtypeStruct((GB, DIM), jnp.float32),
        mesh=_MESH,
        compiler_params=_SC_PARAMS,
        scratch_types=[
            pltpu.VMEM((GSTEPS, GCH), jnp.int32),
            pltpu.VMEM((GCH, DIM), jnp.float32),
            pltpu.SemaphoreType.DMA,
        ],
    )


_gather_kernel = _make_gather()


_RB = 5000


def _scale0_body(deg_ref, e1_ref, dinv_ref, y0_ref):
    deg = deg_ref[:, 0:1]
    dinv = jnp.where(deg > 0, lax.rsqrt(jnp.maximum(deg, 1.0)), 0.0)
    dinv_ref[...] = dinv
    y0_ref[...] = dinv * e1_ref[...]


_scale0 = pl.pallas_call(
    _scale0_body,
    grid=(MN // _RB,),
    in_specs=[pl.BlockSpec((_RB, 16), lambda i: (i, 0)),
              pl.BlockSpec((_RB, DIM), lambda i: (i, 0))],
    out_specs=[pl.BlockSpec((_RB, 1), lambda i: (i, 0)),
               pl.BlockSpec((_RB, DIM), lambda i: (i, 0))],
    out_shape=[jax.ShapeDtypeStruct((MN, 1), jnp.float32),
               jax.ShapeDtypeStruct((MN, DIM), jnp.float32)],
)


def _scale1_body(dinv_ref, t1_ref, y1_ref):
    d = dinv_ref[...]
    y1_ref[...] = d * d * t1_ref[...]


_scale1 = pl.pallas_call(
    _scale1_body,
    grid=(MN // _RB,),
    in_specs=[pl.BlockSpec((_RB, 1), lambda i: (i, 0)),
              pl.BlockSpec((_RB, DIM), lambda i: (i, 0))],
    out_specs=pl.BlockSpec((_RB, DIM), lambda i: (i, 0)),
    out_shape=jax.ShapeDtypeStruct((MN, DIM), jnp.float32),
)


def _combine_body(e1_ref, dinv_ref, t1_ref, t2_ref, emb_ref):
    d = dinv_ref[...]
    emb_ref[...] = (e1_ref[...] + d * (t1_ref[...] + t2_ref[...])) * (1.0 / 3.0)


_combine = pl.pallas_call(
    _combine_body,
    grid=(MN // _RB,),
    in_specs=[pl.BlockSpec((_RB, DIM), lambda i: (i, 0)),
              pl.BlockSpec((_RB, 1), lambda i: (i, 0)),
              pl.BlockSpec((_RB, DIM), lambda i: (i, 0)),
              pl.BlockSpec((_RB, DIM), lambda i: (i, 0))],
    out_specs=pl.BlockSpec((_RB, DIM), lambda i: (i, 0)),
    out_shape=jax.ShapeDtypeStruct((MN, DIM), jnp.float32),
)


def _loss_body(rows_ref, out_ref):
    u_ = rows_ref[0:B, :]
    v_ = rows_ref[B:2 * B, :]
    pos = jnp.sum(u_ * v_, axis=-1, keepdims=True)
    reg = jnp.sum(u_ * u_) + jnp.sum(v_ * v_)
    bpr = jnp.float32(0.0)
    for j in range(NNEG):
        nj = rows_ref[(2 + j) * B:(3 + j) * B, :]
        negs = jnp.sum(u_ * nj, axis=-1, keepdims=True)
        bpr += jnp.sum(jax.nn.log_sigmoid(pos - negs))
        reg += jnp.sum(nj * nj)
    out_ref[...] = (-bpr + REG * reg).reshape(1, 1)


_loss = pl.pallas_call(
    _loss_body,
    out_shape=jax.ShapeDtypeStruct((1, 1), jnp.float32),
)


def kernel(E1, edge_index, u, v, n):
    src = edge_index[0]
    dst = edge_index[1]
    pad = PER_CORE - NE
    gpad = jnp.zeros((pad,), jnp.int32)
    spad = jnp.full((pad,), DUMMY, jnp.int32)
    gidx = jnp.stack([jnp.concatenate([src[:NE], gpad]),
                      jnp.concatenate([src[NE:], gpad])])
    sidx = jnp.stack([jnp.concatenate([dst[:NE] - M, spad]),
                      jnp.concatenate([dst[NE:], spad])])
    gidx = gidx.reshape(NC * NS, CPT, CHUNK)
    sidx = sidx.reshape(NC * NS, CPT, CHUNK)

    degt = _deg_kernel(sidx)
    dinv, y0 = _scale0(degt, E1)
    t1 = _layer_kernel(gidx, sidx, y0)
    y1 = _scale1(dinv, t1)
    t2 = _layer_kernel(gidx, sidx, y1)
    emb = _combine(E1, dinv, t1, t2)

    gather_idx = jnp.concatenate([u, v, n.T.reshape(-1)]).astype(jnp.int32)
    gather_idx = gather_idx.reshape(NC * NS, GSTEPS, GCH)
    rows = _gather_kernel(gather_idx, emb)
    out = _loss(rows)
    return out[0, 0]


# speedup vs baseline: 18.1964x; 1.0434x over previous
import jax
import jax.numpy as jnp
from jax import lax
from jax.experimental import pallas as pl
from jax.experimental.pallas import tpu as pltpu
from jax.experimental.pallas import tpu_sc as plsc

M = 30000
N = 20000
MN = M + N
DIM = 64
NE = 400000
B = 4096
NNEG = 10
REG = 1e-4

NC = 2
NS = 16
CHUNK = 64
G = 8
NG = 49
CPT = G * NG
PER_CORE = NS * CPT * CHUNK
ACC_ROWS = 30016
DUMMY = ACC_ROWS - 1
ZCH = 32
NZ = ACC_ROWS // ZCH
ZSTEPS = (NZ + NS - 1) // NS
CCH = 40
COPY_STEPS = (M // CCH + NS - 1) // NS
WAVE = 16

_MESH = plsc.VectorSubcoreMesh(core_axis_name="c", subcore_axis_name="s")
_SC_PARAMS = pltpu.CompilerParams(use_tc_tiling_on_sc=False)


def _wait(src, dst, sem):
    pltpu.make_async_copy(src, dst, sem).wait()


def _fill(ref, val):
    r_, c_ = ref.shape
    vec = jnp.full((16,), val, jnp.float32)

    def row(r, carry):
        for kk in range(c_ // 16):
            ref[r, pl.ds(kk * 16, 16)] = vec
        return carry

    lax.fori_loop(0, r_, row, 0)


def _zero_acc(s, acc, zsrc, semz):
    for w0 in range(0, ZSTEPS, WAVE):
        wn = min(WAVE, ZSTEPS - w0)
        for phase in range(2):
            for t in range(wn):
                step = w0 + t
                g = step * NS + s

                def op(g=g):
                    if phase == 0:
                        pltpu.async_copy(zsrc, acc.at[pl.ds(g * ZCH, ZCH)],
                                         semz)
                    else:
                        _wait(zsrc, acc.at[pl.ds(0, ZCH)], semz)
                if (step + 1) * NS <= NZ:
                    op()
                else:
                    pl.when(g < NZ)(op)


def _copy_out(c, s, acc, out, semc, width):
    limit = jnp.where(c == 0, N // CCH, M // CCH)
    obase = jnp.where(c == 0, M, 0)
    for w0 in range(0, COPY_STEPS, WAVE):
        wn = min(WAVE, COPY_STEPS - w0)
        for phase in range(2):
            for t in range(wn):
                g = (w0 + t) * NS + s

                def op(g=g):
                    if phase == 0:
                        pltpu.async_copy(acc.at[pl.ds(g * CCH, CCH)],
                                         out.at[pl.ds(obase + g * CCH, CCH)],
                                         semc)
                    else:
                        _wait(acc.at[pl.ds(0, CCH)], out.at[pl.ds(0, CCH)],
                              semc)
                pl.when(g < limit)(op)


def _make_deg():
    def body(sidx, out, sidx_v, obuf, zbuf, accd, semd, semz, semc):
        c = lax.axis_index("c")
        s = lax.axis_index("s")
        w = c * NS + s
        _fill(obuf, 1.0)
        _fill(zbuf, 0.0)
        _zero_acc(s, accd, zbuf, semz)
        pltpu.sync_copy(sidx.at[w], sidx_v)
        plsc.subcore_barrier()

        def grp(g, carry):
            for k in range(G):
                pltpu.async_copy(obuf, accd.at[sidx_v.at[g * G + k]], semd,
                                 add=True)
            for k in range(G):
                _wait(obuf, accd.at[pl.ds(0, CHUNK)], semd)
            return carry

        lax.fori_loop(0, NG, grp, 0)
        plsc.subcore_barrier()
        _copy_out(c, s, accd, out, semc, 16)

    return pl.kernel(
        body,
        out_type=jax.ShapeDtypeStruct((MN, 16), jnp.float32),
        mesh=_MESH,
        compiler_params=_SC_PARAMS,
        scratch_types=[
            pltpu.VMEM((CPT, CHUNK), jnp.int32),
            pltpu.VMEM((CHUNK, 16), jnp.float32),
            pltpu.VMEM((ZCH, 16), jnp.float32),
            pltpu.VMEM_SHARED((ACC_ROWS, 16), jnp.float32),
            pltpu.SemaphoreType.DMA,
            pltpu.SemaphoreType.DMA,
            pltpu.SemaphoreType.DMA,
        ],
    )


_deg_kernel = _make_deg()


def _make_layer():
    def body(gidx, sidx, table, out,
             gbuf, sbuf, rows_v, acc, semg0, semg1, sems0, sems1,
             semi, semz, semc):
        c = lax.axis_index("c")
        s = lax.axis_index("s")
        w = c * NS + s
        semg = (semg0, semg1)
        sems = (sems0, sems1)
        _fill(rows_v.at[0], 0.0)
        _zero_acc(s, acc, rows_v.at[0].at[pl.ds(0, ZCH)], semz)
        pltpu.sync_copy(gidx.at[w].at[pl.ds(0, G)], gbuf.at[0])
        pltpu.sync_copy(sidx.at[w].at[pl.ds(0, G)], sbuf.at[0])
        pltpu.async_copy(table.at[gbuf.at[0].at[0]], rows_v.at[0], semg0)
        plsc.subcore_barrier()

        def group(g, carry):
            cur = g % 2
            nxt = 1 - cur
            gb = gbuf.at[cur]
            sb = sbuf.at[cur]
            for k in range(G):
                j = g * G + k
                b = k % 2
                if k == G - 1:
                    @pl.when(g + 1 < NG)
                    def _():
                        _wait(gidx.at[w].at[pl.ds(0, G)], gbuf.at[0], semi)
                        _wait(gidx.at[w].at[pl.ds(0, G)], sbuf.at[0], semi)
                _wait(table.at[pl.ds(0, CHUNK)], rows_v.at[b], semg[b])
                pltpu.async_copy(rows_v.at[b], acc.at[sb.at[k]], sems[b],
                                 add=True)
                gnext = gb.at[k + 1] if k + 1 < G else gbuf.at[nxt].at[0]
                if k == 0:
                    @pl.when(g >= 1)
                    def _():
                        _wait(rows_v.at[1], acc.at[pl.ds(0, CHUNK)], sems[1])
                    pltpu.async_copy(table.at[gnext], rows_v.at[1], semg1)
                elif k < G - 1:
                    _wait(rows_v.at[1 - b], acc.at[pl.ds(0, CHUNK)],
                          sems[1 - b])
                    pltpu.async_copy(table.at[gnext], rows_v.at[1 - b],
                                     semg[1 - b])
                else:
                    @pl.when(g + 1 < NG)
                    def _():
                        _wait(rows_v.at[0], acc.at[pl.ds(0, CHUNK)], sems[0])
                        pltpu.async_copy(table.at[gnext], rows_v.at[0], semg0)
                if k == 1:
                    @pl.when(g + 1 < NG)
                    def _():
                        pltpu.async_copy(gidx.at[w].at[pl.ds((g + 1) * G, G)],
                                         gbuf.at[nxt], semi)
                        pltpu.async_copy(sidx.at[w].at[pl.ds((g + 1) * G, G)],
                                         sbuf.at[nxt], semi)
            return carry

        lax.fori_loop(0, NG, group, 0)
        _wait(rows_v.at[0], acc.at[pl.ds(0, CHUNK)], sems0)
        _wait(rows_v.at[1], acc.at[pl.ds(0, CHUNK)], sems1)
        plsc.subcore_barrier()
        _copy_out(c, s, acc, out, semc, DIM)

    return pl.kernel(
        body,
        out_type=jax.ShapeDtypeStruct((MN, DIM), jnp.float32),
        mesh=_MESH,
        compiler_params=_SC_PARAMS,
        scratch_types=[
            pltpu.VMEM((2, G, CHUNK), jnp.int32),
            pltpu.VMEM((2, G, CHUNK), jnp.int32),
            pltpu.VMEM((2, CHUNK, DIM), jnp.float32),
            pltpu.VMEM_SHARED((ACC_ROWS, DIM), jnp.float32),
            pltpu.SemaphoreType.DMA,
            pltpu.SemaphoreType.DMA,
            pltpu.SemaphoreType.DMA,
            pltpu.SemaphoreType.DMA,
            pltpu.SemaphoreType.DMA,
            pltpu.SemaphoreType.DMA,
            pltpu.SemaphoreType.DMA,
        ],
    )


_layer_kernel = _make_layer()


GB = B * (NNEG + 2)
GPW = GB // (NC * NS)
GCH = 128
GSTEPS = GPW // GCH


def _make_gather():
    def body(idx, emb, out, idx_v, rbuf, semg0, semg1, semw0, semw1):
        c = lax.axis_index("c")
        s = lax.axis_index("s")
        w = c * NS + s
        semg = (semg0, semg1)
        semw = (semw0, semw1)
        pltpu.sync_copy(idx.at[w], idx_v)
        pltpu.async_copy(emb.at[idx_v.at[0]], rbuf.at[0], semg0)
        for j in range(GSTEPS):
            b = j % 2
            if j + 1 < GSTEPS:
                if j >= 1:
                    _wait(rbuf.at[1 - b], out.at[pl.ds(0, GCH)], semw[1 - b])
                pltpu.async_copy(emb.at[idx_v.at[j + 1]], rbuf.at[1 - b],
                                 semg[1 - b])
            _wait(emb.at[pl.ds(0, GCH)], rbuf.at[b], semg[b])
            pltpu.async_copy(rbuf.at[b],
                             out.at[pl.ds(w * GPW + j * GCH, GCH)], semw[b])
        _wait(rbuf.at[0], out.at[pl.ds(0, GCH)], semw0)
        _wait(rbuf.at[1], out.at[pl.ds(0, GCH)], semw1)

    return pl.kernel(
        body,
        out_type=jax.ShapeDtypeStruct((GB, DIM), jnp.float32),
        mesh=_MESH,
        compiler_params=_SC_PARAMS,
        scratch_types=[
            pltpu.VMEM((GSTEPS, GCH), jnp.int32),
            pltpu.VMEM((2, GCH, DIM), jnp.float32),
            pltpu.SemaphoreType.DMA,
            pltpu.SemaphoreType.DMA,
            pltpu.SemaphoreType.DMA,
            pltpu.SemaphoreType.DMA,
        ],
    )


_gather_kernel = _make_gather()


_RB = 5000


def _scale0_body(deg_ref, e1_ref, dinv_ref, y0_ref):
    deg = deg_ref[:, 0:1]
    dinv = jnp.where(deg > 0, lax.rsqrt(jnp.maximum(deg, 1.0)), 0.0)
    dinv_ref[...] = dinv
    y0_ref[...] = dinv * e1_ref[...]


_scale0 = pl.pallas_call(
    _scale0_body,
    grid=(MN // _RB,),
    in_specs=[pl.BlockSpec((_RB, 16), lambda i: (i, 0)),
              pl.BlockSpec((_RB, DIM), lambda i: (i, 0))],
    out_specs=[pl.BlockSpec((_RB, 1), lambda i: (i, 0)),
               pl.BlockSpec((_RB, DIM), lambda i: (i, 0))],
    out_shape=[jax.ShapeDtypeStruct((MN, 1), jnp.float32),
               jax.ShapeDtypeStruct((MN, DIM), jnp.float32)],
)


def _scale1_body(dinv_ref, t1_ref, y1_ref):
    d = dinv_ref[...]
    y1_ref[...] = d * d * t1_ref[...]


_scale1 = pl.pallas_call(
    _scale1_body,
    grid=(MN // _RB,),
    in_specs=[pl.BlockSpec((_RB, 1), lambda i: (i, 0)),
              pl.BlockSpec((_RB, DIM), lambda i: (i, 0))],
    out_specs=pl.BlockSpec((_RB, DIM), lambda i: (i, 0)),
    out_shape=jax.ShapeDtypeStruct((MN, DIM), jnp.float32),
)


def _combine_body(e1_ref, dinv_ref, t1_ref, t2_ref, emb_ref):
    d = dinv_ref[...]
    emb_ref[...] = (e1_ref[...] + d * (t1_ref[...] + t2_ref[...])) * (1.0 / 3.0)


_combine = pl.pallas_call(
    _combine_body,
    grid=(MN // _RB,),
    in_specs=[pl.BlockSpec((_RB, DIM), lambda i: (i, 0)),
              pl.BlockSpec((_RB, 1), lambda i: (i, 0)),
              pl.BlockSpec((_RB, DIM), lambda i: (i, 0)),
              pl.BlockSpec((_RB, DIM), lambda i: (i, 0))],
    out_specs=pl.BlockSpec((_RB, DIM), lambda i: (i, 0)),
    out_shape=jax.ShapeDtypeStruct((MN, DIM), jnp.float32),
)


def _loss_body(rows_ref, out_ref):
    u_ = rows_ref[0:B, :]
    v_ = rows_ref[B:2 * B, :]
    pos = jnp.sum(u_ * v_, axis=-1, keepdims=True)
    reg = jnp.sum(u_ * u_) + jnp.sum(v_ * v_)
    bpr = jnp.float32(0.0)
    for j in range(NNEG):
        nj = rows_ref[(2 + j) * B:(3 + j) * B, :]
        negs = jnp.sum(u_ * nj, axis=-1, keepdims=True)
        bpr += jnp.sum(jax.nn.log_sigmoid(pos - negs))
        reg += jnp.sum(nj * nj)
    out_ref[...] = (-bpr + REG * reg).reshape(1, 1)


_loss = pl.pallas_call(
    _loss_body,
    out_shape=jax.ShapeDtypeStruct((1, 1), jnp.float32),
)


def kernel(E1, edge_index, u, v, n):
    src = edge_index[0]
    dst = edge_index[1]
    pad = PER_CORE - NE
    gpad = jnp.zeros((pad,), jnp.int32)
    spad = jnp.full((pad,), DUMMY, jnp.int32)
    gidx = jnp.stack([jnp.concatenate([src[:NE], gpad]),
                      jnp.concatenate([src[NE:], gpad])])
    sidx = jnp.stack([jnp.concatenate([dst[:NE] - M, spad]),
                      jnp.concatenate([dst[NE:], spad])])
    gidx = gidx.reshape(NC * NS, CPT, CHUNK)
    sidx = sidx.reshape(NC * NS, CPT, CHUNK)

    degt = _deg_kernel(sidx)
    dinv, y0 = _scale0(degt, E1)
    t1 = _layer_kernel(gidx, sidx, y0)
    y1 = _scale1(dinv, t1)
    t2 = _layer_kernel(gidx, sidx, y1)
    emb = _combine(E1, dinv, t1, t2)

    gather_idx = jnp.concatenate([u, v, n.T.reshape(-1)]).astype(jnp.int32)
    gather_idx = gather_idx.reshape(NC * NS, GSTEPS, GCH)
    rows = _gather_kernel(gather_idx, emb)
    out = _loss(rows)
    return out[0, 0]


# speedup vs baseline: 20.6638x; 1.1356x over previous
import jax
import jax.numpy as jnp
from jax import lax
from jax.experimental import pallas as pl
from jax.experimental.pallas import tpu as pltpu
from jax.experimental.pallas import tpu_sc as plsc

M = 30000
N = 20000
MN = M + N
DIM = 64
NE = 400000
B = 4096
NNEG = 10
REG = 1e-4

NC = 2
NS = 16
CHUNK = 64
G = 8
NG = 49
CPT = G * NG
PER_CORE = NS * CPT * CHUNK
ACC_ROWS = 30016
DUMMY = ACC_ROWS - 1
ZCH = 32
NZ = ACC_ROWS // ZCH
ZSTEPS = (NZ + NS - 1) // NS
CCH = 40
COPY_STEPS = (M // CCH + NS - 1) // NS
WAVE = 16

_MESH = plsc.VectorSubcoreMesh(core_axis_name="c", subcore_axis_name="s")
_SC_PARAMS = pltpu.CompilerParams(use_tc_tiling_on_sc=False)


def _wait(src, dst, sem):
    pltpu.make_async_copy(src, dst, sem).wait()


def _fill(ref, val):
    r_, c_ = ref.shape
    vec = jnp.full((16,), val, jnp.float32)

    def row(r, carry):
        for kk in range(c_ // 16):
            ref[r, pl.ds(kk * 16, 16)] = vec
        return carry

    lax.fori_loop(0, r_, row, 0)


def _zero_acc(s, acc, zsrc, semz):
    for w0 in range(0, ZSTEPS, WAVE):
        wn = min(WAVE, ZSTEPS - w0)
        for phase in range(2):
            for t in range(wn):
                step = w0 + t
                g = step * NS + s

                def op(g=g):
                    if phase == 0:
                        pltpu.async_copy(zsrc, acc.at[pl.ds(g * ZCH, ZCH)],
                                         semz)
                    else:
                        _wait(zsrc, acc.at[pl.ds(0, ZCH)], semz)
                if (step + 1) * NS <= NZ:
                    op()
                else:
                    pl.when(g < NZ)(op)


def _copy_out(c, s, acc, out, semc, width):
    limit = jnp.where(c == 0, N // CCH, M // CCH)
    obase = jnp.where(c == 0, M, 0)
    for w0 in range(0, COPY_STEPS, WAVE):
        wn = min(WAVE, COPY_STEPS - w0)
        for phase in range(2):
            for t in range(wn):
                g = (w0 + t) * NS + s

                def op(g=g):
                    if phase == 0:
                        pltpu.async_copy(acc.at[pl.ds(g * CCH, CCH)],
                                         out.at[pl.ds(obase + g * CCH, CCH)],
                                         semc)
                    else:
                        _wait(acc.at[pl.ds(0, CCH)], out.at[pl.ds(0, CCH)],
                              semc)
                pl.when(g < limit)(op)


_RSQRT_MAGIC = 0x5F3759DF


def _rsqrt16(x):
    i = plsc.bitcast(x, jnp.int32)
    i = jnp.full((16,), _RSQRT_MAGIC, jnp.int32) - lax.shift_right_logical(i, 1)
    y = plsc.bitcast(i, jnp.float32)
    for _ in range(3):
        y = y * (1.5 - 0.5 * x * y * y)
    return y


def _make_deg():
    def body(sidx, out, sidx_v, obuf, zbuf, dbuf, wbuf, accd,
             semd, semz, semc):
        c = lax.axis_index("c")
        s = lax.axis_index("s")
        w = c * NS + s
        _fill(obuf, 1.0)
        _fill(zbuf, 0.0)
        _zero_acc(s, accd, zbuf, semz)
        pltpu.sync_copy(sidx.at[w], sidx_v)
        plsc.subcore_barrier()

        def grp(g, carry):
            for k in range(G):
                pltpu.async_copy(obuf, accd.at[sidx_v.at[g * G + k]], semd,
                                 add=True)
            for k in range(G):
                _wait(obuf, accd.at[pl.ds(0, CHUNK)], semd)
            return carry

        lax.fori_loop(0, NG, grp, 0)
        plsc.subcore_barrier()

        limit = jnp.where(c == 0, N // CCH, M // CCH)
        obase = jnp.where(c == 0, M, 0)

        def step(jj, carry):
            g = jj * NS + s

            @pl.when(g < limit)
            def _():
                pltpu.sync_copy(accd.at[pl.ds(g * CCH, CCH)], dbuf)

                def row(r, cc):
                    deg = dbuf[r, pl.ds(0, 16)]
                    dv = jnp.where(deg > 0.0,
                                   _rsqrt16(jnp.maximum(deg, 1.0)), 0.0)
                    for kk in range(DIM // 16):
                        wbuf[r, pl.ds(kk * 16, 16)] = dv
                    return cc

                lax.fori_loop(0, CCH, row, 0)
                pltpu.sync_copy(wbuf, out.at[pl.ds(obase + g * CCH, CCH)])
            return carry

        lax.fori_loop(0, COPY_STEPS, step, 0)

    return pl.kernel(
        body,
        out_type=jax.ShapeDtypeStruct((MN, DIM), jnp.float32),
        mesh=_MESH,
        compiler_params=pltpu.CompilerParams(use_tc_tiling_on_sc=False,
                                             needs_layout_passes=False),
        scratch_types=[
            pltpu.VMEM((CPT, CHUNK), jnp.int32),
            pltpu.VMEM((CHUNK, 16), jnp.float32),
            pltpu.VMEM((ZCH, 16), jnp.float32),
            pltpu.VMEM((CCH, 16), jnp.float32),
            pltpu.VMEM((CCH, DIM), jnp.float32),
            pltpu.VMEM_SHARED((ACC_ROWS, 16), jnp.float32),
            pltpu.SemaphoreType.DMA,
            pltpu.SemaphoreType.DMA,
            pltpu.SemaphoreType.DMA,
        ],
    )


_deg_kernel = _make_deg()


def _make_layer():
    def body(gidx, sidx, table, out,
             gbuf, sbuf, rows_v, acc, semg0, semg1, sems0, sems1,
             semi, semz, semc):
        c = lax.axis_index("c")
        s = lax.axis_index("s")
        w = c * NS + s
        semg = (semg0, semg1)
        sems = (sems0, sems1)
        _fill(rows_v.at[0], 0.0)
        _zero_acc(s, acc, rows_v.at[0].at[pl.ds(0, ZCH)], semz)
        pltpu.sync_copy(gidx.at[w].at[pl.ds(0, G)], gbuf.at[0])
        pltpu.sync_copy(sidx.at[w].at[pl.ds(0, G)], sbuf.at[0])
        pltpu.async_copy(table.at[gbuf.at[0].at[0]], rows_v.at[0], semg0)
        plsc.subcore_barrier()

        def group(g, carry):
            cur = g % 2
            nxt = 1 - cur
            gb = gbuf.at[cur]
            sb = sbuf.at[cur]
            for k in range(G):
                j = g * G + k
                b = k % 2
                if k == G - 1:
                    @pl.when(g + 1 < NG)
                    def _():
                        _wait(gidx.at[w].at[pl.ds(0, G)], gbuf.at[0], semi)
                        _wait(gidx.at[w].at[pl.ds(0, G)], sbuf.at[0], semi)
                _wait(table.at[pl.ds(0, CHUNK)], rows_v.at[b], semg[b])
                pltpu.async_copy(rows_v.at[b], acc.at[sb.at[k]], sems[b],
                                 add=True)
                gnext = gb.at[k + 1] if k + 1 < G else gbuf.at[nxt].at[0]
                if k == 0:
                    @pl.when(g >= 1)
                    def _():
                        _wait(rows_v.at[1], acc.at[pl.ds(0, CHUNK)], sems[1])
                    pltpu.async_copy(table.at[gnext], rows_v.at[1], semg1)
                elif k < G - 1:
                    _wait(rows_v.at[1 - b], acc.at[pl.ds(0, CHUNK)],
                          sems[1 - b])
                    pltpu.async_copy(table.at[gnext], rows_v.at[1 - b],
                                     semg[1 - b])
                else:
                    @pl.when(g + 1 < NG)
                    def _():
                        _wait(rows_v.at[0], acc.at[pl.ds(0, CHUNK)], sems[0])
                        pltpu.async_copy(table.at[gnext], rows_v.at[0], semg0)
                if k == 1:
                    @pl.when(g + 1 < NG)
                    def _():
                        pltpu.async_copy(gidx.at[w].at[pl.ds((g + 1) * G, G)],
                                         gbuf.at[nxt], semi)
                        pltpu.async_copy(sidx.at[w].at[pl.ds((g + 1) * G, G)],
                                         sbuf.at[nxt], semi)
            return carry

        lax.fori_loop(0, NG, group, 0)
        _wait(rows_v.at[0], acc.at[pl.ds(0, CHUNK)], sems0)
        _wait(rows_v.at[1], acc.at[pl.ds(0, CHUNK)], sems1)
        plsc.subcore_barrier()
        _copy_out(c, s, acc, out, semc, DIM)

    return pl.kernel(
        body,
        out_type=jax.ShapeDtypeStruct((MN, DIM), jnp.float32),
        mesh=_MESH,
        compiler_params=_SC_PARAMS,
        scratch_types=[
            pltpu.VMEM((2, G, CHUNK), jnp.int32),
            pltpu.VMEM((2, G, CHUNK), jnp.int32),
            pltpu.VMEM((2, CHUNK, DIM), jnp.float32),
            pltpu.VMEM_SHARED((ACC_ROWS, DIM), jnp.float32),
            pltpu.SemaphoreType.DMA,
            pltpu.SemaphoreType.DMA,
            pltpu.SemaphoreType.DMA,
            pltpu.SemaphoreType.DMA,
            pltpu.SemaphoreType.DMA,
            pltpu.SemaphoreType.DMA,
            pltpu.SemaphoreType.DMA,
        ],
    )


_layer_kernel = _make_layer()


GB = B * (NNEG + 2)
GPW = GB // (NC * NS)
GCH = 128
GSTEPS = GPW // GCH


def _make_gather():
    def body(idx, emb, out, idx_v, rbuf, semg0, semg1, semw0, semw1):
        c = lax.axis_index("c")
        s = lax.axis_index("s")
        w = c * NS + s
        semg = (semg0, semg1)
        semw = (semw0, semw1)
        pltpu.sync_copy(idx.at[w], idx_v)
        pltpu.async_copy(emb.at[idx_v.at[0]], rbuf.at[0], semg0)
        for j in range(GSTEPS):
            b = j % 2
            if j + 1 < GSTEPS:
                if j >= 1:
                    _wait(rbuf.at[1 - b], out.at[pl.ds(0, GCH)], semw[1 - b])
                pltpu.async_copy(emb.at[idx_v.at[j + 1]], rbuf.at[1 - b],
                                 semg[1 - b])
            _wait(emb.at[pl.ds(0, GCH)], rbuf.at[b], semg[b])
            pltpu.async_copy(rbuf.at[b],
                             out.at[pl.ds(w * GPW + j * GCH, GCH)], semw[b])
        _wait(rbuf.at[0], out.at[pl.ds(0, GCH)], semw0)
        _wait(rbuf.at[1], out.at[pl.ds(0, GCH)], semw1)

    return pl.kernel(
        body,
        out_type=jax.ShapeDtypeStruct((GB, DIM), jnp.float32),
        mesh=_MESH,
        compiler_params=_SC_PARAMS,
        scratch_types=[
            pltpu.VMEM((GSTEPS, GCH), jnp.int32),
            pltpu.VMEM((2, GCH, DIM), jnp.float32),
            pltpu.SemaphoreType.DMA,
            pltpu.SemaphoreType.DMA,
            pltpu.SemaphoreType.DMA,
            pltpu.SemaphoreType.DMA,
        ],
    )


_gather_kernel = _make_gather()


_FL = MN * DIM
_FB = 640000
_FGRID = _FL // _FB


def _scale0_body(dw_ref, e1_ref, y0_ref):
    y0_ref[...] = dw_ref[...] * e1_ref[...]


_scale0 = pl.pallas_call(
    _scale0_body,
    grid=(_FGRID,),
    in_specs=[pl.BlockSpec((_FB,), lambda i: (i,)),
              pl.BlockSpec((_FB,), lambda i: (i,))],
    out_specs=pl.BlockSpec((_FB,), lambda i: (i,)),
    out_shape=jax.ShapeDtypeStruct((_FL,), jnp.float32),
)


def _scale1_body(dw_ref, t1_ref, y1_ref):
    d = dw_ref[...]
    y1_ref[...] = d * d * t1_ref[...]


_scale1 = pl.pallas_call(
    _scale1_body,
    grid=(_FGRID,),
    in_specs=[pl.BlockSpec((_FB,), lambda i: (i,)),
              pl.BlockSpec((_FB,), lambda i: (i,))],
    out_specs=pl.BlockSpec((_FB,), lambda i: (i,)),
    out_shape=jax.ShapeDtypeStruct((_FL,), jnp.float32),
)


def _combine_body(e1_ref, dw_ref, t1_ref, t2_ref, emb_ref):
    emb_ref[...] = (e1_ref[...] + dw_ref[...] * (t1_ref[...] + t2_ref[...])) * (1.0 / 3.0)


_combine = pl.pallas_call(
    _combine_body,
    grid=(_FGRID,),
    in_specs=[pl.BlockSpec((_FB,), lambda i: (i,)),
              pl.BlockSpec((_FB,), lambda i: (i,)),
              pl.BlockSpec((_FB,), lambda i: (i,)),
              pl.BlockSpec((_FB,), lambda i: (i,))],
    out_specs=pl.BlockSpec((_FB,), lambda i: (i,)),
    out_shape=jax.ShapeDtypeStruct((_FL,), jnp.float32),
)


def _loss_body(rows_ref, out_ref):
    u_ = rows_ref[0:B, :]
    v_ = rows_ref[B:2 * B, :]
    pos = jnp.sum(u_ * v_, axis=-1, keepdims=True)
    reg = jnp.sum(u_ * u_) + jnp.sum(v_ * v_)
    bpr = jnp.float32(0.0)
    for j in range(NNEG):
        nj = rows_ref[(2 + j) * B:(3 + j) * B, :]
        negs = jnp.sum(u_ * nj, axis=-1, keepdims=True)
        bpr += jnp.sum(jax.nn.log_sigmoid(pos - negs))
        reg += jnp.sum(nj * nj)
    out_ref[...] = (-bpr + REG * reg).reshape(1, 1)


_loss = pl.pallas_call(
    _loss_body,
    out_shape=jax.ShapeDtypeStruct((1, 1), jnp.float32),
)


def kernel(E1, edge_index, u, v, n):
    src = edge_index[0]
    dst = edge_index[1]
    pad = PER_CORE - NE
    gpad = jnp.zeros((pad,), jnp.int32)
    spad = jnp.full((pad,), DUMMY, jnp.int32)
    gidx = jnp.stack([jnp.concatenate([src[:NE], gpad]),
                      jnp.concatenate([src[NE:], gpad])])
    sidx = jnp.stack([jnp.concatenate([dst[:NE] - M, spad]),
                      jnp.concatenate([dst[NE:], spad])])
    gidx = gidx.reshape(NC * NS, CPT, CHUNK)
    sidx = sidx.reshape(NC * NS, CPT, CHUNK)

    dwf = _deg_kernel(sidx).reshape(-1)
    e1f = E1.reshape(-1)
    y0 = _scale0(dwf, e1f).reshape(MN, DIM)
    t1 = _layer_kernel(gidx, sidx, y0)
    t1f = t1.reshape(-1)
    y1 = _scale1(dwf, t1f).reshape(MN, DIM)
    t2 = _layer_kernel(gidx, sidx, y1)
    embf = _combine(e1f, dwf, t1f, t2.reshape(-1))
    emb = embf.reshape(MN, DIM)

    gather_idx = jnp.concatenate([u, v, n.T.reshape(-1)]).astype(jnp.int32)
    gather_idx = gather_idx.reshape(NC * NS, GSTEPS, GCH)
    rows = _gather_kernel(gather_idx, emb)
    out = _loss(rows)
    return out[0, 0]


# speedup vs baseline: 20.9304x; 1.0129x over previous
import jax
import jax.numpy as jnp
from jax import lax
from jax.experimental import pallas as pl
from jax.experimental.pallas import tpu as pltpu
from jax.experimental.pallas import tpu_sc as plsc

M = 30000
N = 20000
MN = M + N
DIM = 64
NE = 400000
B = 4096
NNEG = 10
REG = 1e-4

NC = 2
NS = 16
CHUNK = 64
G = 8
NG = 49
CPT = G * NG
PER_CORE = NS * CPT * CHUNK
ACC_ROWS = 30016
DUMMY = ACC_ROWS - 1
ZCH = 32
NZ = ACC_ROWS // ZCH
ZSTEPS = (NZ + NS - 1) // NS
CCH = 40
COPY_STEPS = (M // CCH + NS - 1) // NS
WAVE = 16

_MESH = plsc.VectorSubcoreMesh(core_axis_name="c", subcore_axis_name="s")
_SC_PARAMS = pltpu.CompilerParams(use_tc_tiling_on_sc=False)


def _wait(src, dst, sem):
    pltpu.make_async_copy(src, dst, sem).wait()


def _fill(ref, val):
    r_, c_ = ref.shape
    vec = jnp.full((16,), val, jnp.float32)

    def row(r, carry):
        for kk in range(c_ // 16):
            ref[r, pl.ds(kk * 16, 16)] = vec
        return carry

    lax.fori_loop(0, r_, row, 0)


def _zero_acc(s, acc, zsrc, semz):
    for w0 in range(0, ZSTEPS, WAVE):
        wn = min(WAVE, ZSTEPS - w0)
        for phase in range(2):
            for t in range(wn):
                step = w0 + t
                g = step * NS + s

                def op(g=g):
                    if phase == 0:
                        pltpu.async_copy(zsrc, acc.at[pl.ds(g * ZCH, ZCH)],
                                         semz)
                    else:
                        _wait(zsrc, acc.at[pl.ds(0, ZCH)], semz)
                if (step + 1) * NS <= NZ:
                    op()
                else:
                    pl.when(g < NZ)(op)


def _copy_out(c, s, acc, out, semc, width):
    limit = jnp.where(c == 0, N // CCH, M // CCH)
    obase = jnp.where(c == 0, M, 0)
    for w0 in range(0, COPY_STEPS, WAVE):
        wn = min(WAVE, COPY_STEPS - w0)
        for phase in range(2):
            for t in range(wn):
                g = (w0 + t) * NS + s

                def op(g=g):
                    if phase == 0:
                        pltpu.async_copy(acc.at[pl.ds(g * CCH, CCH)],
                                         out.at[pl.ds(obase + g * CCH, CCH)],
                                         semc)
                    else:
                        _wait(acc.at[pl.ds(0, CCH)], out.at[pl.ds(0, CCH)],
                              semc)
                pl.when(g < limit)(op)


_RSQRT_MAGIC = 0x5F3759DF


def _rsqrt16(x):
    i = plsc.bitcast(x, jnp.int32)
    i = jnp.full((16,), _RSQRT_MAGIC, jnp.int32) - lax.shift_right_logical(i, 1)
    y = plsc.bitcast(i, jnp.float32)
    for _ in range(3):
        y = y * (1.5 - 0.5 * x * y * y)
    return y


DCH = 80
DSTEPS = (M // DCH + NS - 1) // NS


def _make_deg():
    def body(sidx, out, sidx_v, obuf, zbuf, dbuf, wbuf, accd,
             semd, semz, semw0, semw1):
        c = lax.axis_index("c")
        s = lax.axis_index("s")
        w = c * NS + s
        semw = (semw0, semw1)
        _fill(obuf, 1.0)
        _fill(zbuf, 0.0)
        _zero_acc(s, accd, zbuf, semz)
        pltpu.sync_copy(sidx.at[w], sidx_v)
        plsc.subcore_barrier()

        def grp(g, carry):
            for k in range(G):
                pltpu.async_copy(obuf, accd.at[sidx_v.at[g * G + k]], semd,
                                 add=True)
            for k in range(G):
                _wait(obuf, accd.at[pl.ds(0, CHUNK)], semd)
            return carry

        lax.fori_loop(0, NG, grp, 0)
        plsc.subcore_barrier()

        limit = jnp.where(c == 0, N // DCH, M // DCH)
        obase = jnp.where(c == 0, M, 0)
        for step in range(DSTEPS):
            b = step % 2
            g = step * NS + s

            def op(b=b, g=g, step=step):
                if step >= 2:
                    _wait(wbuf.at[b], out.at[pl.ds(0, DCH)], semw[b])
                pltpu.sync_copy(accd.at[pl.ds(g * DCH, DCH)], dbuf.at[b])

                def row(r, cc):
                    deg = dbuf[b, r, pl.ds(0, 16)]
                    dv = jnp.where(deg > 0.0,
                                   _rsqrt16(jnp.maximum(deg, 1.0)), 0.0)
                    for kk in range(DIM // 16):
                        wbuf[b, r, pl.ds(kk * 16, 16)] = dv
                    return cc

                lax.fori_loop(0, DCH, row, 0)
                pltpu.async_copy(wbuf.at[b],
                                 out.at[pl.ds(obase + g * DCH, DCH)], semw[b])
            pl.when(g < limit)(op)
        _wait(wbuf.at[0], out.at[pl.ds(0, DCH)], semw[0])
        _wait(wbuf.at[1], out.at[pl.ds(0, DCH)], semw[1])

    return pl.kernel(
        body,
        out_type=jax.ShapeDtypeStruct((MN, DIM), jnp.float32),
        mesh=_MESH,
        compiler_params=pltpu.CompilerParams(use_tc_tiling_on_sc=False,
                                             needs_layout_passes=False),
        scratch_types=[
            pltpu.VMEM((CPT, CHUNK), jnp.int32),
            pltpu.VMEM((CHUNK, 16), jnp.float32),
            pltpu.VMEM((ZCH, 16), jnp.float32),
            pltpu.VMEM((2, DCH, 16), jnp.float32),
            pltpu.VMEM((2, DCH, DIM), jnp.float32),
            pltpu.VMEM_SHARED((ACC_ROWS, 16), jnp.float32),
            pltpu.SemaphoreType.DMA,
            pltpu.SemaphoreType.DMA,
            pltpu.SemaphoreType.DMA,
            pltpu.SemaphoreType.DMA,
        ],
    )


_deg_kernel = _make_deg()


def _make_layer():
    def body(gidx, sidx, table, out,
             gbuf, sbuf, rows_v, acc, semg0, semg1, sems0, sems1,
             semi, semz, semc):
        c = lax.axis_index("c")
        s = lax.axis_index("s")
        w = c * NS + s
        semg = (semg0, semg1)
        sems = (sems0, sems1)
        _fill(rows_v.at[0], 0.0)
        _zero_acc(s, acc, rows_v.at[0].at[pl.ds(0, ZCH)], semz)
        pltpu.sync_copy(gidx.at[w].at[pl.ds(0, G)], gbuf.at[0])
        pltpu.sync_copy(sidx.at[w].at[pl.ds(0, G)], sbuf.at[0])
        pltpu.async_copy(table.at[gbuf.at[0].at[0]], rows_v.at[0], semg0)
        plsc.subcore_barrier()

        def group(g, carry):
            cur = g % 2
            nxt = 1 - cur
            gb = gbuf.at[cur]
            sb = sbuf.at[cur]
            for k in range(G):
                j = g * G + k
                b = k % 2
                if k == G - 1:
                    @pl.when(g + 1 < NG)
                    def _():
                        _wait(gidx.at[w].at[pl.ds(0, G)], gbuf.at[0], semi)
                        _wait(gidx.at[w].at[pl.ds(0, G)], sbuf.at[0], semi)
                _wait(table.at[pl.ds(0, CHUNK)], rows_v.at[b], semg[b])
                pltpu.async_copy(rows_v.at[b], acc.at[sb.at[k]], sems[b],
                                 add=True)
                gnext = gb.at[k + 1] if k + 1 < G else gbuf.at[nxt].at[0]
                if k == 0:
                    @pl.when(g >= 1)
                    def _():
                        _wait(rows_v.at[1], acc.at[pl.ds(0, CHUNK)], sems[1])
                    pltpu.async_copy(table.at[gnext], rows_v.at[1], semg1)
                elif k < G - 1:
                    _wait(rows_v.at[1 - b], acc.at[pl.ds(0, CHUNK)],
                          sems[1 - b])
                    pltpu.async_copy(table.at[gnext], rows_v.at[1 - b],
                                     semg[1 - b])
                else:
                    @pl.when(g + 1 < NG)
                    def _():
                        _wait(rows_v.at[0], acc.at[pl.ds(0, CHUNK)], sems[0])
                        pltpu.async_copy(table.at[gnext], rows_v.at[0], semg0)
                if k == 1:
                    @pl.when(g + 1 < NG)
                    def _():
                        pltpu.async_copy(gidx.at[w].at[pl.ds((g + 1) * G, G)],
                                         gbuf.at[nxt], semi)
                        pltpu.async_copy(sidx.at[w].at[pl.ds((g + 1) * G, G)],
                                         sbuf.at[nxt], semi)
            return carry

        lax.fori_loop(0, NG, group, 0)
        _wait(rows_v.at[0], acc.at[pl.ds(0, CHUNK)], sems0)
        _wait(rows_v.at[1], acc.at[pl.ds(0, CHUNK)], sems1)
        plsc.subcore_barrier()
        _copy_out(c, s, acc, out, semc, DIM)

    return pl.kernel(
        body,
        out_type=jax.ShapeDtypeStruct((MN, DIM), jnp.float32),
        mesh=_MESH,
        compiler_params=_SC_PARAMS,
        scratch_types=[
            pltpu.VMEM((2, G, CHUNK), jnp.int32),
            pltpu.VMEM((2, G, CHUNK), jnp.int32),
            pltpu.VMEM((2, CHUNK, DIM), jnp.float32),
            pltpu.VMEM_SHARED((ACC_ROWS, DIM), jnp.float32),
            pltpu.SemaphoreType.DMA,
            pltpu.SemaphoreType.DMA,
            pltpu.SemaphoreType.DMA,
            pltpu.SemaphoreType.DMA,
            pltpu.SemaphoreType.DMA,
            pltpu.SemaphoreType.DMA,
            pltpu.SemaphoreType.DMA,
        ],
    )


_layer_kernel = _make_layer()


NROW = NNEG + 2
SPW = B // (NC * NS)
SCOL = 16 * (NNEG + 2)
NK = DIM // 16


def _make_gather():
    def body(idx, emb, out, idx_v, rbuf, sbuf, semg):
        c = lax.axis_index("c")
        s = lax.axis_index("s")
        w = c * NS + s
        pltpu.sync_copy(idx.at[w], idx_v)
        for j in range(NROW):
            pltpu.async_copy(emb.at[idx_v.at[j]], rbuf.at[j], semg)
        for j in range(NROW):
            _wait(emb.at[pl.ds(0, SPW)], rbuf.at[0], semg)

        def sample(i, carry):
            uv = [rbuf[0, i, pl.ds(kk * 16, 16)] for kk in range(NK)]
            vv = [rbuf[1, i, pl.ds(kk * 16, 16)] for kk in range(NK)]
            pos = uv[0] * vv[0]
            reg = uv[0] * uv[0] + vv[0] * vv[0]
            for kk in range(1, NK):
                pos = pos + uv[kk] * vv[kk]
                reg = reg + uv[kk] * uv[kk] + vv[kk] * vv[kk]
            sbuf[i, pl.ds(0, 16)] = pos
            for j in range(NNEG):
                nv = [rbuf[2 + j, i, pl.ds(kk * 16, 16)] for kk in range(NK)]
                neg = uv[0] * nv[0]
                for kk in range(1, NK):
                    neg = neg + uv[kk] * nv[kk]
                for kk in range(NK):
                    reg = reg + nv[kk] * nv[kk]
                sbuf[i, pl.ds(16 + 16 * j, 16)] = neg
            sbuf[i, pl.ds(16 * (NNEG + 1), 16)] = reg
            return carry

        lax.fori_loop(0, SPW, sample, 0)
        pltpu.sync_copy(sbuf, out.at[w])

    return pl.kernel(
        body,
        out_type=jax.ShapeDtypeStruct((NC * NS, SPW, SCOL), jnp.float32),
        mesh=_MESH,
        compiler_params=pltpu.CompilerParams(use_tc_tiling_on_sc=False,
                                             needs_layout_passes=False),
        scratch_types=[
            pltpu.VMEM((NROW, SPW), jnp.int32),
            pltpu.VMEM((NROW, SPW, DIM), jnp.float32),
            pltpu.VMEM((SPW, SCOL), jnp.float32),
            pltpu.SemaphoreType.DMA,
        ],
    )


_gather_kernel = _make_gather()


_FL = MN * DIM
_FB = 640000
_FGRID = _FL // _FB


def _scale0_body(dw_ref, e1_ref, y0_ref):
    y0_ref[...] = dw_ref[...] * e1_ref[...]


_scale0 = pl.pallas_call(
    _scale0_body,
    grid=(_FGRID,),
    in_specs=[pl.BlockSpec((_FB,), lambda i: (i,)),
              pl.BlockSpec((_FB,), lambda i: (i,))],
    out_specs=pl.BlockSpec((_FB,), lambda i: (i,)),
    out_shape=jax.ShapeDtypeStruct((_FL,), jnp.float32),
)


def _scale1_body(dw_ref, t1_ref, y1_ref):
    d = dw_ref[...]
    y1_ref[...] = d * d * t1_ref[...]


_scale1 = pl.pallas_call(
    _scale1_body,
    grid=(_FGRID,),
    in_specs=[pl.BlockSpec((_FB,), lambda i: (i,)),
              pl.BlockSpec((_FB,), lambda i: (i,))],
    out_specs=pl.BlockSpec((_FB,), lambda i: (i,)),
    out_shape=jax.ShapeDtypeStruct((_FL,), jnp.float32),
)


def _combine_body(e1_ref, dw_ref, t1_ref, t2_ref, emb_ref):
    emb_ref[...] = (e1_ref[...] + dw_ref[...] * (t1_ref[...] + t2_ref[...])) * (1.0 / 3.0)


_combine = pl.pallas_call(
    _combine_body,
    grid=(_FGRID,),
    in_specs=[pl.BlockSpec((_FB,), lambda i: (i,)),
              pl.BlockSpec((_FB,), lambda i: (i,)),
              pl.BlockSpec((_FB,), lambda i: (i,)),
              pl.BlockSpec((_FB,), lambda i: (i,))],
    out_specs=pl.BlockSpec((_FB,), lambda i: (i,)),
    out_shape=jax.ShapeDtypeStruct((_FL,), jnp.float32),
)


def _loss_body(sc_ref, out_ref):
    pos = jnp.sum(sc_ref[:, 0:16], axis=-1, keepdims=True)
    reg = jnp.sum(sc_ref[:, 16 * (NNEG + 1):16 * (NNEG + 2)])
    bpr = jnp.float32(0.0)
    for j in range(NNEG):
        negs = jnp.sum(sc_ref[:, 16 + 16 * j:32 + 16 * j], axis=-1,
                       keepdims=True)
        bpr += jnp.sum(jax.nn.log_sigmoid(pos - negs))
    out_ref[...] = (-bpr + REG * reg).reshape(1, 1)


_loss = pl.pallas_call(
    _loss_body,
    out_shape=jax.ShapeDtypeStruct((1, 1), jnp.float32),
)


def kernel(E1, edge_index, u, v, n):
    src = edge_index[0]
    dst = edge_index[1]
    pad = PER_CORE - NE
    gpad = jnp.zeros((pad,), jnp.int32)
    spad = jnp.full((pad,), DUMMY, jnp.int32)
    gidx = jnp.stack([jnp.concatenate([src[:NE], gpad]),
                      jnp.concatenate([src[NE:], gpad])])
    sidx = jnp.stack([jnp.concatenate([dst[:NE] - M, spad]),
                      jnp.concatenate([dst[NE:], spad])])
    gidx = gidx.reshape(NC * NS, CPT, CHUNK)
    sidx = sidx.reshape(NC * NS, CPT, CHUNK)

    dwf = _deg_kernel(sidx).reshape(-1)
    e1f = E1.reshape(-1)
    y0 = _scale0(dwf, e1f).reshape(MN, DIM)
    t1 = _layer_kernel(gidx, sidx, y0)
    t1f = t1.reshape(-1)
    y1 = _scale1(dwf, t1f).reshape(MN, DIM)
    t2 = _layer_kernel(gidx, sidx, y1)
    embf = _combine(e1f, dwf, t1f, t2.reshape(-1))
    emb = embf.reshape(MN, DIM)

    gather_idx = jnp.concatenate([u[None, :], v[None, :], n.T],
                                 axis=0).astype(jnp.int32)
    gather_idx = gather_idx.reshape(NROW, NC * NS, SPW).transpose(1, 0, 2)
    scores = _gather_kernel(gather_idx, emb)
    out = _loss(scores.reshape(B, SCOL))
    return out[0, 0]
